# baseline (speedup 1.0000x reference)
import functools

import jax
import jax.numpy as jnp
from jax import lax
from jax.experimental import pallas as pl
from jax.experimental.pallas import tpu as pltpu
from jax.experimental.pallas import tpu_sc as plsc

N = 10000
D = 128
H = 64
T = 28
Q = 3
K = 50

NC = 2
NS = 16
L = 16
NW = NC * NS

NP = 10240
DP = 144
F = 96
TP = 32

EB = 128
NB = 80
BLK = 512


def _edge_body(src_hbm, dst_hbm, xp_hbm, agg_hbm,
               sidx_v, didx_v, rows_v, agg_sh,
               semi0, semi1, semi2, semi3, semg0, semg1, sems0, sems1,
               *, epw):
    cid = lax.axis_index("c")
    sid = lax.axis_index("s")
    wid = sid * NC + cid
    nbat = epw // EB
    semi = (semi0, semi1, semi2, semi3)
    semg = (semg0, semg1)
    sems = (sems0, sems1)

    def zrow(i, _):
        for c in range(DP // L):
            rows_v[0, i, pl.ds(c * L, L)] = jnp.zeros((L,), jnp.float32)
        return 0
    lax.fori_loop(0, EB, zrow, 0)
    zrows = NP // NS
    for z in range(zrows // EB):
        pltpu.sync_copy(rows_v.at[0],
                        agg_sh.at[pl.ds(sid * zrows + z * EB, EB)])
    plsc.subcore_barrier()

    e0 = wid * epw

    def fire_idx(j, q):
        pltpu.async_copy(src_hbm.at[pl.ds(e0 + j * EB, EB)],
                         sidx_v.at[q], semi[q])
        pltpu.async_copy(dst_hbm.at[pl.ds(e0 + j * EB, EB)],
                         didx_v.at[q], semi[q])

    def wait_idx(q):
        pltpu.make_async_copy(src_hbm.at[pl.ds(e0, EB)],
                              sidx_v.at[q], semi[q]).wait()
        pltpu.make_async_copy(dst_hbm.at[pl.ds(e0, EB)],
                              didx_v.at[q], semi[q]).wait()

    fire_idx(0, 0)
    fire_idx(1, 1)
    fire_idx(2, 2)
    wait_idx(0)
    pltpu.async_copy(xp_hbm.at[sidx_v.at[0]], rows_v.at[0], semg[0])

    def outer(jo, _):
        for u in range(4):
            j = jo * 4 + u
            b = u % 2
            q = u

            @pl.when(j >= 1)
            def _():
                pltpu.make_async_copy(rows_v.at[1 - b],
                                      agg_sh.at[pl.ds(0, EB)],
                                      sems[1 - b]).wait()

            @pl.when(j + 3 < nbat)
            def _():
                fire_idx(j + 3, (q + 3) % 4)

            @pl.when(j + 1 < nbat)
            def _():
                wait_idx((q + 1) % 4)
            pltpu.make_async_copy(xp_hbm.at[sidx_v.at[q]], rows_v.at[b],
                                  semg[b]).wait()

            @pl.when(j + 1 < nbat)
            def _():
                pltpu.async_copy(xp_hbm.at[sidx_v.at[(q + 1) % 4]],
                                 rows_v.at[1 - b], semg[1 - b])
            pltpu.async_copy(rows_v.at[b], agg_sh.at[didx_v.at[q]],
                             sems[b], add=True)
        return 0
    lax.fori_loop(0, nbat // 4, outer, 0)
    pltpu.make_async_copy(rows_v.at[(nbat - 1) % 2], agg_sh.at[pl.ds(0, EB)],
                          sems[(nbat - 1) % 2]).wait()
    plsc.subcore_barrier()

    pltpu.sync_copy(agg_sh.at[pl.ds(sid * zrows, zrows)],
                    agg_hbm.at[cid, pl.ds(sid * zrows, zrows)])


def _dense_body(xp_ref, agg_ref, sc_ref, msk_ref, valid_ref,
                ws_ref, wn_ref, wp_ref, b_ref, r_ref,
                wt_ref, base_ref, w_ref):
    a = agg_ref[0] + agg_ref[1]
    deg = jnp.maximum(a[:, D:D + 1], 1.0)
    agg = a[:, :D] / deg
    xb = xp_ref[...][:, :D]
    h = jnp.maximum(xb @ ws_ref[...] + agg @ wn_ref[...], 0.0)
    out96 = h @ wp_ref[...] + b_ref[...]
    sc = sc_ref[...]
    scb = sc @ r_ref[...]
    inv = (1.0 / sc) @ r_ref[...]
    m = msk_ref[...] > 0.0
    wt_ref[...] = out96 * scb * valid_ref[...]
    base_ref[...] = jnp.where(m, 0.0, out96)
    w_ref[...] = jnp.where(m, inv, 0.0)


def _kb_body(kbt_hbm, wt_hbm, base_hbm, w_hbm, out_hbm,
             kb_v, acc_v, bb_v, ww_v, semk, semg0, semg1, semb0, semb1):
    cid = lax.axis_index("c")
    sid = lax.axis_index("s")
    wid = sid * NC + cid
    npw = NP // NW
    nbat = npw // NB
    n0 = wid * npw
    b0 = n0 // NB
    semg = (semg0, semg1)
    semb = (semb0, semb1)

    def zacc(i, _):
        for bz in range(2):
            for c in range(F // L):
                acc_v[bz, i, pl.ds(c * L, L)] = jnp.zeros((L,), jnp.float32)
        return 0
    lax.fori_loop(0, NB, zacc, 0)

    def load_idx(jj, b):
        pltpu.async_copy(kbt_hbm.at[b0 + jj], kb_v.at[b], semk)

    def wait_idx(b):
        pltpu.make_async_copy(kbt_hbm.at[b0], kb_v.at[b], semk).wait()

    def fire_bw(jj, b):
        nb0 = n0 + jj * NB
        pltpu.async_copy(base_hbm.at[pl.ds(nb0, NB)], bb_v.at[b], semb[b])
        pltpu.async_copy(w_hbm.at[pl.ds(nb0, NB)], ww_v.at[b], semb[b])

    def fire_adds(b):
        def kf(k, _):
            pltpu.async_copy(wt_hbm.at[kb_v.at[b, k]], acc_v.at[b],
                             semg[b], add=True)
            return 0
        lax.fori_loop(0, K, kf, 0)

    def drain_adds(b):
        def kd(k, _):
            pltpu.make_async_copy(wt_hbm.at[kb_v.at[b, 0]], acc_v.at[b],
                                  semg[b]).wait()
            return 0
        lax.fori_loop(0, K, kd, 0)

    load_idx(0, 0)
    fire_bw(0, 0)
    wait_idx(0)
    if nbat > 1:
        load_idx(1, 1)
    fire_adds(0)

    for jj in range(nbat):
        b = jj % 2
        nb0 = n0 + jj * NB
        if jj + 1 < nbat:
            wait_idx(1 - b)
            fire_bw(jj + 1, 1 - b)
            fire_adds(1 - b)
        drain_adds(b)
        if jj + 2 < nbat:
            load_idx(jj + 2, b)
        pltpu.make_async_copy(base_hbm.at[pl.ds(nb0, NB)], bb_v.at[b],
                              semb[b]).wait()
        pltpu.make_async_copy(w_hbm.at[pl.ds(nb0, NB)], ww_v.at[b],
                              semb[b]).wait()

        def comb(i, _):
            for c in range(F // L):
                s = pl.ds(c * L, L)
                acc_v[b, i, s] = bb_v[b, i, s] + acc_v[b, i, s] * ww_v[b, i, s]
            return 0
        lax.fori_loop(0, NB, comb, 0)
        pltpu.sync_copy(acc_v.at[b], out_hbm.at[pl.ds(nb0, NB)])
        if jj + 2 < nbat:
            def zr(i, _):
                for c in range(F // L):
                    acc_v[b, i, pl.ds(c * L, L)] = jnp.zeros((L,),
                                                             jnp.float32)
                return 0
            lax.fori_loop(0, NB, zr, 0)


def kernel(x, edge_index, keybom, scaler, key_aggregation_status,
           W_self, W_neigh, W_proj, b_proj):
    f32 = jnp.float32
    i32 = jnp.int32
    E = edge_index.shape[1]
    epw = -(-E // (NW * 4 * EB)) * 4 * EB
    EP = epw * NW

    xp = jnp.zeros((NP, DP), f32).at[:N, :D].set(x).at[:N, D].set(1.0)
    srcp = jnp.full((EP,), N, i32).at[:E].set(edge_index[0])
    dstp = jnp.full((EP,), N, i32).at[:E].set(edge_index[1])
    kb = jnp.where(keybom < 0, N, keybom)
    kbt3 = (jnp.full((K, NP), N, i32).at[:, :N].set(kb.T)
            .reshape(K, NP // NB, NB).transpose(1, 0, 2))
    scp = jnp.ones((NP, TP), f32).at[:N, :T].set(scaler)
    mskf = jnp.zeros((NP, 1), f32).at[:N].set(
        (key_aggregation_status > 0).astype(f32))
    validf = jnp.zeros((NP, 1), f32).at[:N, :].set(1.0)
    wp96 = jnp.zeros((H, F), f32).at[:, :T * Q].set(W_proj)
    b96 = jnp.zeros((1, F), f32).at[0, :T * Q].set(b_proj)
    rmat = (jnp.arange(F)[None, :] // Q == jnp.arange(TP)[:, None]).astype(f32)

    mesh = plsc.VectorSubcoreMesh(core_axis_name="c", subcore_axis_name="s",
                                  num_cores=NC, num_subcores=NS)

    edge_fn = pl.kernel(
        functools.partial(_edge_body, epw=epw),
        out_type=jax.ShapeDtypeStruct((NC, NP, DP), f32),
        mesh=mesh,
        compiler_params=pltpu.CompilerParams(use_tc_tiling_on_sc=False),
        scratch_types=[
            pltpu.VMEM((4, EB), i32),
            pltpu.VMEM((4, EB), i32),
            pltpu.VMEM((2, EB, DP), f32),
            pltpu.VMEM_SHARED((NP, DP), f32),
        ] + [pltpu.SemaphoreType.DMA] * 8,
    )
    agg2 = edge_fn(srcp, dstp, xp)

    grid = NP // BLK
    wt, base, w = pl.pallas_call(
        _dense_body,
        grid=(grid,),
        in_specs=[
            pl.BlockSpec((BLK, DP), lambda i: (i, 0)),
            pl.BlockSpec((NC, BLK, DP), lambda i: (0, i, 0)),
            pl.BlockSpec((BLK, TP), lambda i: (i, 0)),
            pl.BlockSpec((BLK, 1), lambda i: (i, 0)),
            pl.BlockSpec((BLK, 1), lambda i: (i, 0)),
            pl.BlockSpec((D, H), lambda i: (0, 0)),
            pl.BlockSpec((D, H), lambda i: (0, 0)),
            pl.BlockSpec((H, F), lambda i: (0, 0)),
            pl.BlockSpec((1, F), lambda i: (0, 0)),
            pl.BlockSpec((TP, F), lambda i: (0, 0)),
        ],
        out_specs=[
            pl.BlockSpec((BLK, F), lambda i: (i, 0)),
            pl.BlockSpec((BLK, F), lambda i: (i, 0)),
            pl.BlockSpec((BLK, F), lambda i: (i, 0)),
        ],
        out_shape=[
            jax.ShapeDtypeStruct((NP, F), f32),
            jax.ShapeDtypeStruct((NP, F), f32),
            jax.ShapeDtypeStruct((NP, F), f32),
        ],
    )(xp, agg2, scp, mskf, validf, W_self, W_neigh, wp96, b96, rmat)

    kb_fn = pl.kernel(
        _kb_body,
        out_type=jax.ShapeDtypeStruct((NP, F), f32),
        mesh=mesh,
        compiler_params=pltpu.CompilerParams(use_tc_tiling_on_sc=False),
        scratch_types=[
            pltpu.VMEM((2, K, NB), i32),
            pltpu.VMEM((2, NB, F), f32),
            pltpu.VMEM((2, NB, F), f32),
            pltpu.VMEM((2, NB, F), f32),
        ] + [pltpu.SemaphoreType.DMA] * 5,
    )
    outp = kb_fn(kbt3, wt, base, w)

    return outp[:N, :T * Q].reshape(N, T, Q)


# speedup vs baseline: 1.1000x; 1.1000x over previous
import functools

import jax
import jax.numpy as jnp
from jax import lax
from jax.experimental import pallas as pl
from jax.experimental.pallas import tpu as pltpu
from jax.experimental.pallas import tpu_sc as plsc

N = 10000
D = 128
H = 64
T = 28
Q = 3
K = 50

NC = 2
NS = 16
L = 16
NW = NC * NS

NP = 10240
DP = 144
F = 96
TP = 32

EB = 128
NB = 80
BLK = 512

EBAT0 = 104
EBAT1 = 56
KBAT0 = 7
KBAT1 = 1


def _edge_body(src_hbm, dst_hbm, xp_hbm, agg_hbm,
               sidx_v, didx_v, rows_v, agg_sh, semi, semg, sems):
    cid = lax.axis_index("c")
    sid = lax.axis_index("s")
    nbat = jnp.where(cid == 0, EBAT0, EBAT1)
    bat0 = jnp.where(cid == 0, sid * EBAT0, NS * EBAT0 + sid * EBAT1)

    def zrow(i, _):
        for c in range(DP // L):
            rows_v[0, i, pl.ds(c * L, L)] = jnp.zeros((L,), jnp.float32)
        return 0
    lax.fori_loop(0, EB, zrow, 0)
    zrows = NP // NS
    for z in range(zrows // EB):
        pltpu.sync_copy(rows_v.at[0],
                        agg_sh.at[pl.ds(sid * zrows + z * EB, EB)])
    plsc.subcore_barrier()

    e0 = bat0 * EB
    pltpu.async_copy(src_hbm.at[pl.ds(e0, EB)], sidx_v.at[0], semi)
    pltpu.async_copy(dst_hbm.at[pl.ds(e0, EB)], didx_v.at[0], semi)

    def body(j, _):
        b = j % 2
        base = e0 + j * EB
        pltpu.make_async_copy(src_hbm.at[pl.ds(base, EB)],
                              sidx_v.at[b], semi).wait()
        pltpu.make_async_copy(dst_hbm.at[pl.ds(base, EB)],
                              didx_v.at[b], semi).wait()

        @pl.when(j + 1 < nbat)
        def _():
            pltpu.async_copy(src_hbm.at[pl.ds(base + EB, EB)],
                             sidx_v.at[1 - b], semi)
            pltpu.async_copy(dst_hbm.at[pl.ds(base + EB, EB)],
                             didx_v.at[1 - b], semi)

        @pl.when(j >= 2)
        def _():
            pltpu.make_async_copy(rows_v.at[b],
                                  agg_sh.at[pl.ds(0, EB)], sems).wait()
        pltpu.async_copy(xp_hbm.at[sidx_v.at[b]], rows_v.at[b], semg).wait()
        pltpu.async_copy(rows_v.at[b], agg_sh.at[didx_v.at[b]], sems,
                         add=True)
        return 0
    lax.fori_loop(0, nbat, body, 0)
    pltpu.make_async_copy(rows_v.at[0], agg_sh.at[pl.ds(0, EB)], sems).wait()
    pltpu.make_async_copy(rows_v.at[1], agg_sh.at[pl.ds(0, EB)], sems).wait()
    plsc.subcore_barrier()

    pltpu.sync_copy(agg_sh.at[pl.ds(sid * zrows, zrows)],
                    agg_hbm.at[cid, pl.ds(sid * zrows, zrows)])


def _dense_body(xp_ref, agg_ref, sc_ref, msk_ref, valid_ref,
                ws_ref, wn_ref, wp_ref, b_ref, r_ref,
                wt_ref, base_ref, w_ref):
    a = agg_ref[0] + agg_ref[1]
    deg = jnp.maximum(a[:, D:D + 1], 1.0)
    agg = a[:, :D] / deg
    xb = xp_ref[...][:, :D]
    h = jnp.maximum(xb @ ws_ref[...] + agg @ wn_ref[...], 0.0)
    out96 = h @ wp_ref[...] + b_ref[...]
    sc = sc_ref[...]
    scb = sc @ r_ref[...]
    inv = (1.0 / sc) @ r_ref[...]
    m = msk_ref[...] > 0.0
    wt_ref[...] = out96 * scb * valid_ref[...]
    base_ref[...] = jnp.where(m, 0.0, out96)
    w_ref[...] = jnp.where(m, inv, 0.0)


def _kb_body(kbt_hbm, wt_hbm, base_hbm, w_hbm, out_hbm,
             kb_v, acc_v, bb_v, ww_v, sem, sem2):
    cid = lax.axis_index("c")
    sid = lax.axis_index("s")
    nbat = jnp.where(cid == 0, KBAT0, KBAT1)
    blk0 = jnp.where(cid == 0, sid * KBAT0, NS * KBAT0 + sid * KBAT1)

    def batch(j, _):
        bidx = blk0 + j
        nb = bidx * NB
        pltpu.sync_copy(kbt_hbm.at[bidx], kb_v)
        cb = pltpu.async_copy(base_hbm.at[pl.ds(nb, NB)], bb_v, sem2)
        cw = pltpu.async_copy(w_hbm.at[pl.ds(nb, NB)], ww_v, sem2)
        pltpu.async_copy(wt_hbm.at[kb_v.at[0]], acc_v, sem).wait()

        W = 8

        def kfire(k, _):
            pltpu.async_copy(wt_hbm.at[kb_v.at[k]], acc_v, sem, add=True)

            @pl.when(k >= W + 1)
            def _():
                pltpu.make_async_copy(wt_hbm.at[kb_v.at[0]], acc_v,
                                      sem).wait()
            return 0
        lax.fori_loop(1, K, kfire, 0)

        def kdrain(k, _):
            pltpu.make_async_copy(wt_hbm.at[kb_v.at[0]], acc_v, sem).wait()
            return 0
        lax.fori_loop(0, W, kdrain, 0)
        cb.wait()
        cw.wait()

        def comb(i, _):
            for c in range(F // L):
                s = pl.ds(c * L, L)
                acc_v[i, s] = bb_v[i, s] + acc_v[i, s] * ww_v[i, s]
            return 0
        lax.fori_loop(0, NB, comb, 0)
        pltpu.sync_copy(acc_v, out_hbm.at[pl.ds(nb, NB)])
        return 0
    lax.fori_loop(0, nbat, batch, 0)


def kernel(x, edge_index, keybom, scaler, key_aggregation_status,
           W_self, W_neigh, W_proj, b_proj):
    f32 = jnp.float32
    i32 = jnp.int32
    E = edge_index.shape[1]
    EP = NS * (EBAT0 + EBAT1) * EB
    assert EP >= E

    xp = jnp.zeros((NP, DP), f32).at[:N, :D].set(x).at[:N, D].set(1.0)
    srcp = jnp.full((EP,), N, i32).at[:E].set(edge_index[0])
    dstp = jnp.full((EP,), N, i32).at[:E].set(edge_index[1])
    kb = jnp.where(keybom < 0, N, keybom)
    kbt3 = (jnp.full((K, NP), N, i32).at[:, :N].set(kb.T)
            .reshape(K, NP // NB, NB).transpose(1, 0, 2))
    scp = jnp.ones((NP, TP), f32).at[:N, :T].set(scaler)
    mskf = jnp.zeros((NP, 1), f32).at[:N].set(
        (key_aggregation_status > 0).astype(f32))
    validf = jnp.zeros((NP, 1), f32).at[:N, :].set(1.0)
    wp96 = jnp.zeros((H, F), f32).at[:, :T * Q].set(W_proj)
    b96 = jnp.zeros((1, F), f32).at[0, :T * Q].set(b_proj)
    rmat = (jnp.arange(F)[None, :] // Q == jnp.arange(TP)[:, None]).astype(f32)

    mesh = plsc.VectorSubcoreMesh(core_axis_name="c", subcore_axis_name="s",
                                  num_cores=NC, num_subcores=NS)

    edge_fn = pl.kernel(
        _edge_body,
        out_type=jax.ShapeDtypeStruct((NC, NP, DP), f32),
        mesh=mesh,
        compiler_params=pltpu.CompilerParams(use_tc_tiling_on_sc=False),
        scratch_types=[
            pltpu.VMEM((2, EB), i32),
            pltpu.VMEM((2, EB), i32),
            pltpu.VMEM((2, EB, DP), f32),
            pltpu.VMEM_SHARED((NP, DP), f32),
            pltpu.SemaphoreType.DMA,
            pltpu.SemaphoreType.DMA,
            pltpu.SemaphoreType.DMA,
        ],
    )
    agg2 = edge_fn(srcp, dstp, xp)

    grid = NP // BLK
    wt, base, w = pl.pallas_call(
        _dense_body,
        grid=(grid,),
        in_specs=[
            pl.BlockSpec((BLK, DP), lambda i: (i, 0)),
            pl.BlockSpec((NC, BLK, DP), lambda i: (0, i, 0)),
            pl.BlockSpec((BLK, TP), lambda i: (i, 0)),
            pl.BlockSpec((BLK, 1), lambda i: (i, 0)),
            pl.BlockSpec((BLK, 1), lambda i: (i, 0)),
            pl.BlockSpec((D, H), lambda i: (0, 0)),
            pl.BlockSpec((D, H), lambda i: (0, 0)),
            pl.BlockSpec((H, F), lambda i: (0, 0)),
            pl.BlockSpec((1, F), lambda i: (0, 0)),
            pl.BlockSpec((TP, F), lambda i: (0, 0)),
        ],
        out_specs=[
            pl.BlockSpec((BLK, F), lambda i: (i, 0)),
            pl.BlockSpec((BLK, F), lambda i: (i, 0)),
            pl.BlockSpec((BLK, F), lambda i: (i, 0)),
        ],
        out_shape=[
            jax.ShapeDtypeStruct((NP, F), f32),
            jax.ShapeDtypeStruct((NP, F), f32),
            jax.ShapeDtypeStruct((NP, F), f32),
        ],
    )(xp, agg2, scp, mskf, validf, W_self, W_neigh, wp96, b96, rmat)

    kb_fn = pl.kernel(
        _kb_body,
        out_type=jax.ShapeDtypeStruct((NP, F), f32),
        mesh=mesh,
        compiler_params=pltpu.CompilerParams(use_tc_tiling_on_sc=False),
        scratch_types=[
            pltpu.VMEM((K, NB), i32),
            pltpu.VMEM((NB, F), f32),
            pltpu.VMEM((NB, F), f32),
            pltpu.VMEM((NB, F), f32),
            pltpu.SemaphoreType.DMA,
            pltpu.SemaphoreType.DMA,
        ],
    )
    outp = kb_fn(kbt3, wt, base, w)

    return outp[:N, :T * Q].reshape(N, T, Q)
